# baseline (speedup 1.0000x reference)
import functools

import jax
import jax.numpy as jnp
from jax import lax
from jax.experimental import pallas as pl
from jax.experimental.pallas import tpu as pltpu
from jax.experimental.pallas import tpu_sc as plsc

R, C, D, B = 1000, 8, 128, 16384
GW = 2 * C * D + 16
LOG_OFF = 2 * C * D

NC, NS, L = 2, 16, 16
NW = NC * NS
BPW = B // NW
NE = 16
NCH = BPW // NE
DG = D // L

_LN2 = 0.6931471805599453


def _vlog(x):
  bits = plsc.bitcast(x, jnp.int32)
  e = jnp.right_shift(bits, 23) - 127
  m = plsc.bitcast(
      jnp.bitwise_or(jnp.bitwise_and(bits, 0x007FFFFF), 0x3F800000),
      jnp.float32)
  big = m > 1.4142135623730951
  m = jnp.where(big, m * 0.5, m)
  e = jnp.where(big, e + 1, e)
  s = (m - 1.0) / (m + 1.0)
  s2 = s * s
  p = 2.0 * s * (1.0 + s2 * (1.0 / 3.0 + s2 * (0.2 + s2 * (1.0 / 7.0))))
  return e.astype(jnp.float32) * _LN2 + p


def _sc_fused(idx, tab):
  mesh = plsc.VectorSubcoreMesh(core_axis_name="c", subcore_axis_name="s")

  @functools.partial(
      pl.kernel,
      out_type=[
          jax.ShapeDtypeStruct((B, D), jnp.float32),
          jax.ShapeDtypeStruct((B, D), jnp.float32),
      ],
      mesh=mesh,
      compiler_params=pltpu.CompilerParams(use_tc_tiling_on_sc=False,
                                           needs_layout_passes=False),
      scratch_types=[
          pltpu.VMEM((BPW,), jnp.int32),
          pltpu.VMEM((NE, GW), jnp.float32),
          pltpu.VMEM((NE, GW), jnp.float32),
          pltpu.VMEM((NE, D), jnp.float32),
          pltpu.VMEM((NE, D), jnp.float32),
          pltpu.VMEM((NE, D), jnp.float32),
          pltpu.VMEM((NE, D), jnp.float32),
          pltpu.SemaphoreType.DMA,
          pltpu.SemaphoreType.DMA,
          pltpu.SemaphoreType.DMA,
          pltpu.SemaphoreType.DMA,
      ],
  )
  def k(idx_hbm, tab_hbm, mu_hbm, lv_hbm,
        idx_v, buf0, buf1, omu0, olv0, omu1, olv1, sem0, sem1, wsem0, wsem1):
    wid = lax.axis_index("s") * NC + lax.axis_index("c")
    base = wid * BPW
    pltpu.sync_copy(idx_hbm.at[pl.ds(base, BPW)], idx_v)

    bufs = (buf0, buf1)
    sems = (sem0, sem1)

    def start_gather(ci, slot):
      pltpu.async_copy(
          tab_hbm.at[idx_v.at[pl.ds(ci * NE, NE)]], bufs[slot], sems[slot])

    def wait_gather(slot):
      pltpu.make_async_copy(
          tab_hbm.at[idx_v.at[pl.ds(0, NE)]], bufs[slot], sems[slot]).wait()

    rows = lax.iota(jnp.int32, L)
    outs = ((omu0, olv0, wsem0), (omu1, olv1, wsem1))

    def compute(ci, buf, slot):
      omu, olv, wsem = outs[slot]
      @pl.when(ci >= 2)
      def _():
        pltpu.make_async_copy(
            omu, mu_hbm.at[pl.ds(base, NE)], wsem).wait()
        pltpu.make_async_copy(
            olv, lv_hbm.at[pl.ds(base, NE)], wsem).wait()

      ls = [plsc.load_gather(buf, [rows, jnp.full((L,), LOG_OFF + c,
                                                  jnp.int32)])
            for c in range(C)]
      mx = ls[0]
      for c in range(1, C):
        mx = jnp.maximum(mx, ls[c])
      es = [jnp.exp(l - mx) for l in ls]
      tot = es[0]
      for c in range(1, C):
        tot = tot + es[c]
      inv = 1.0 / tot
      ws = [e * inv for e in es]

      def one_d(dvec):
        mu_a = jnp.zeros((L,), jnp.float32)
        mu_b = jnp.zeros((L,), jnp.float32)
        sm_a = jnp.zeros((L,), jnp.float32)
        sm_b = jnp.zeros((L,), jnp.float32)
        for c in range(C):
          mu_cd = plsc.load_gather(
              buf, [rows, jnp.full((L,), c * D, jnp.int32) + dvec])
          lv_cd = plsc.load_gather(
              buf, [rows, jnp.full((L,), C * D + c * D, jnp.int32) + dvec])
          t = jnp.exp(lv_cd) + mu_cd * mu_cd
          if c % 2 == 0:
            mu_a = mu_a + ws[c] * mu_cd
            sm_a = sm_a + ws[c] * t
          else:
            mu_b = mu_b + ws[c] * mu_cd
            sm_b = sm_b + ws[c] * t
        mu_acc = mu_a + mu_b
        var = jnp.maximum(sm_a + sm_b - mu_acc * mu_acc, 1e-6)
        plsc.store_scatter(omu, [rows, dvec], mu_acc)
        plsc.store_scatter(olv, [rows, dvec], _vlog(var))

      def dstep(dd, carry):
        d0 = jnp.bitwise_and(rows + dd * 2, D - 1)
        one_d(d0)
        one_d(jnp.bitwise_and(d0 + 1, D - 1))
        return carry

      lax.fori_loop(0, D // 2, dstep, 0)
      pltpu.async_copy(omu, mu_hbm.at[pl.ds(base + ci * NE, NE)], wsem)
      pltpu.async_copy(olv, lv_hbm.at[pl.ds(base + ci * NE, NE)], wsem)

    start_gather(0, 0)
    start_gather(1, 1)

    def outer(oi, carry):
      for b in range(2):
        ci = oi * 2 + b
        wait_gather(b)
        compute(ci, bufs[b], b)

        @pl.when(ci + 2 < NCH)
        def _():
          start_gather(ci + 2, b)
      return carry

    lax.fori_loop(0, NCH // 2, outer, 0)
    for s in range(2):
      omu, olv, wsem = outs[s]
      pltpu.make_async_copy(omu, mu_hbm.at[pl.ds(base, NE)], wsem).wait()
      pltpu.make_async_copy(olv, lv_hbm.at[pl.ds(base, NE)], wsem).wait()

  return k(idx, tab)


def kernel(regime_id, regime_seen_mask, logits_emb, mu_emb, logvar_emb,
           logits_unknown, mu_unknown, logvar_unknown):
  rid = jnp.clip(regime_id, 0, R - 1).astype(jnp.int32)
  idx = jnp.where(regime_seen_mask, rid, R).astype(jnp.int32)
  pad = jnp.full((R + 1, GW - LOG_OFF - C), -1e30, jnp.float32)
  tab = jnp.concatenate([
      jnp.concatenate([mu_emb, mu_unknown.reshape(1, C * D)], 0),
      jnp.concatenate([logvar_emb, logvar_unknown.reshape(1, C * D)], 0),
      jnp.concatenate([logits_emb, logits_unknown.reshape(1, C)], 0),
      pad,
  ], axis=1)
  mu, logvar = _sc_fused(idx, tab)
  return (mu, logvar)


# speedup vs baseline: 2.0267x; 2.0267x over previous
import functools

import jax
import jax.numpy as jnp
from jax import lax
from jax.experimental import pallas as pl
from jax.experimental.pallas import tpu as pltpu
from jax.experimental.pallas import tpu_sc as plsc

R, C, D, B = 1000, 8, 128, 16384
GW = 2 * C * D + 16
LOG_OFF = 2 * C * D
RP = 1024

NC, NS, L = 2, 16, 16
NW = NC * NS
RPW = RP // NW
BPW = B // NW
NE = 16

_LN2 = 0.6931471805599453


def _vlog(x):
  bits = plsc.bitcast(x, jnp.int32)
  e = jnp.right_shift(bits, 23) - 127
  m = plsc.bitcast(
      jnp.bitwise_or(jnp.bitwise_and(bits, 0x007FFFFF), 0x3F800000),
      jnp.float32)
  big = m > 1.4142135623730951
  m = jnp.where(big, m * 0.5, m)
  e = jnp.where(big, e + 1, e)
  s = (m - 1.0) / (m + 1.0)
  s2 = s * s
  p = 2.0 * s * (1.0 + s2 * (1.0 / 3.0 + s2 * (0.2 + s2 * (1.0 / 7.0))))
  return e.astype(jnp.float32) * _LN2 + p


def _sc_fused(idx, tab):
  mesh = plsc.VectorSubcoreMesh(core_axis_name="c", subcore_axis_name="s")

  @functools.partial(
      pl.kernel,
      out_type=[
          jax.ShapeDtypeStruct((B + NE, D), jnp.float32),
          jax.ShapeDtypeStruct((B + NE, D), jnp.float32),
      ],
      mesh=mesh,
      compiler_params=pltpu.CompilerParams(use_tc_tiling_on_sc=False,
                                           needs_layout_passes=False),
      scratch_types=[
          pltpu.VMEM((B,), jnp.int32),
          pltpu.VMEM((RPW, GW), jnp.float32),
          pltpu.VMEM((1, GW), jnp.float32),
          pltpu.VMEM((B + L,), jnp.int32),
          pltpu.VMEM((B + L,), jnp.int32),
          pltpu.VMEM((BPW + L,), jnp.int32),
          pltpu.VMEM((NE, D), jnp.float32),
          pltpu.VMEM((NE, D), jnp.float32),
          pltpu.VMEM((NE, D), jnp.float32),
          pltpu.VMEM((NE, D), jnp.float32),
          pltpu.VMEM((1, L), jnp.int32),
          pltpu.SemaphoreType.DMA,
      ],
  )
  def k(idx_hbm, tab_hbm, mu_hbm, lv_hbm,
        idx_v, ltab, ubuf, blist, rlist, ulist,
        omu, olv, umu, ulv, bidx, sem):
    wid = lax.axis_index("s") * NC + lax.axis_index("c")
    lo = wid * RPW
    hi = jnp.minimum(lo + RPW, R)
    rows = lax.iota(jnp.int32, L)

    pltpu.sync_copy(idx_hbm, idx_v)
    pltpu.sync_copy(tab_hbm.at[pl.ds(lo, RPW)], ltab)
    pltpu.sync_copy(tab_hbm.at[pl.ds(R, 1)], ubuf)

    def scan_seen(i, off):
      v = idx_v[pl.ds(i * L, L)]
      m = jnp.logical_and(v >= lo, v < hi)
      plsc.store_compressed(blist.at[pl.ds(off, L)], rows + i * L, mask=m)
      plsc.store_compressed(rlist.at[pl.ds(off, L)], v - lo, mask=m)
      return off + plsc.all_reduce_population_count(m)[0]

    cnt = lax.fori_loop(0, B // L, scan_seen, 0)
    blist[pl.ds(cnt, L)] = jnp.full((L,), B, jnp.int32)
    rlist[pl.ds(cnt, L)] = jnp.zeros((L,), jnp.int32)

    ubase = wid * BPW
    def scan_unseen(i, off):
      v = idx_v[pl.ds(ubase + i * L, L)]
      m = v == R
      plsc.store_compressed(ulist.at[pl.ds(off, L)], rows + ubase + i * L,
                             mask=m)
      return off + plsc.all_reduce_population_count(m)[0]

    ucnt = lax.fori_loop(0, BPW // L, scan_unseen, 0)
    ulist[pl.ds(ucnt, L)] = jnp.full((L,), B, jnp.int32)

    def moments(src, rl, dst_mu, dst_lv):
      ls = [plsc.load_gather(src, [rl, jnp.full((L,), LOG_OFF + c,
                                                jnp.int32)])
            for c in range(C)]
      mx = ls[0]
      for c in range(1, C):
        mx = jnp.maximum(mx, ls[c])
      es = [jnp.exp(l - mx) for l in ls]
      tot = es[0]
      for c in range(1, C):
        tot = tot + es[c]
      inv = 1.0 / tot
      ws = [e * inv for e in es]

      def dstep(dd, carry):
        dvec = jnp.bitwise_and(rows + dd, D - 1)
        mu_acc = jnp.zeros((L,), jnp.float32)
        sm_acc = jnp.zeros((L,), jnp.float32)
        for c in range(C):
          mu_cd = plsc.load_gather(
              src, [rl, jnp.full((L,), c * D, jnp.int32) + dvec])
          lv_cd = plsc.load_gather(
              src, [rl, jnp.full((L,), C * D + c * D, jnp.int32) + dvec])
          mu_acc = mu_acc + ws[c] * mu_cd
          sm_acc = sm_acc + ws[c] * (jnp.exp(lv_cd) + mu_cd * mu_cd)
        var = jnp.maximum(sm_acc - mu_acc * mu_acc, 1e-6)
        plsc.store_scatter(dst_mu, [rows, dvec], mu_acc)
        plsc.store_scatter(dst_lv, [rows, dvec], _vlog(var))
        return carry

      lax.fori_loop(0, D, dstep, 0)

    moments(ubuf, jnp.zeros((L,), jnp.int32), umu, ulv)

    def scatter(dst_mu, dst_lv, pos_list, j):
      bidx[0] = pos_list[pl.ds(j * L, L)]
      pltpu.async_copy(dst_mu, mu_hbm.at[bidx.at[0]], sem)
      pltpu.async_copy(dst_lv, lv_hbm.at[bidx.at[0]], sem)
      pltpu.make_async_copy(dst_mu, mu_hbm.at[bidx.at[0]], sem).wait()
      pltpu.make_async_copy(dst_lv, lv_hbm.at[bidx.at[0]], sem).wait()

    def seen_chunk(j, carry):
      rl = rlist[pl.ds(j * L, L)]
      moments(ltab, rl, omu, olv)
      scatter(omu, olv, blist, j)
      return carry

    lax.fori_loop(0, (cnt + L - 1) // L, seen_chunk, 0)

    def unseen_chunk(j, carry):
      scatter(umu, ulv, ulist, j)
      return carry

    lax.fori_loop(0, (ucnt + L - 1) // L, unseen_chunk, 0)

  return k(idx, tab)


def kernel(regime_id, regime_seen_mask, logits_emb, mu_emb, logvar_emb,
           logits_unknown, mu_unknown, logvar_unknown):
  rid = jnp.clip(regime_id, 0, R - 1).astype(jnp.int32)
  idx = jnp.where(regime_seen_mask, rid, R).astype(jnp.int32)
  pad = jnp.full((R + 1, GW - LOG_OFF - C), -1e30, jnp.float32)
  tab = jnp.concatenate([
      jnp.concatenate([mu_emb, mu_unknown.reshape(1, C * D)], 0),
      jnp.concatenate([logvar_emb, logvar_unknown.reshape(1, C * D)], 0),
      jnp.concatenate([logits_emb, logits_unknown.reshape(1, C)], 0),
      pad,
  ], axis=1)
  tab = jnp.concatenate(
      [tab, jnp.zeros((RP - (R + 1), GW), jnp.float32)], axis=0)
  mu, logvar = _sc_fused(idx, tab)
  return (mu[:B], logvar[:B])


# speedup vs baseline: 2.0730x; 1.0228x over previous
import functools

import jax
import jax.numpy as jnp
from jax import lax
from jax.experimental import pallas as pl
from jax.experimental.pallas import tpu as pltpu
from jax.experimental.pallas import tpu_sc as plsc

R, C, D, B = 1000, 8, 128, 16384
GW = 2 * C * D + 16
LOG_OFF = 2 * C * D
RP = 1024

NC, NS, L = 2, 16, 16
NW = NC * NS
RPW = RP // NW
BPW = B // NW
NE = 16

_LN2 = 0.6931471805599453


def _vlog(x):
  bits = plsc.bitcast(x, jnp.int32)
  e = jnp.right_shift(bits, 23) - 127
  m = plsc.bitcast(
      jnp.bitwise_or(jnp.bitwise_and(bits, 0x007FFFFF), 0x3F800000),
      jnp.float32)
  big = m > 1.4142135623730951
  m = jnp.where(big, m * 0.5, m)
  e = jnp.where(big, e + 1, e)
  s = (m - 1.0) / (m + 1.0)
  s2 = s * s
  p = 2.0 * s * (1.0 + s2 * (1.0 / 3.0 + s2 * (0.2 + s2 * (1.0 / 7.0))))
  return e.astype(jnp.float32) * _LN2 + p


def _sc_fused(idx, tab):
  mesh = plsc.VectorSubcoreMesh(core_axis_name="c", subcore_axis_name="s")

  @functools.partial(
      pl.kernel,
      out_type=[
          jax.ShapeDtypeStruct((B + NE, D), jnp.float32),
          jax.ShapeDtypeStruct((B + NE, D), jnp.float32),
      ],
      mesh=mesh,
      compiler_params=pltpu.CompilerParams(use_tc_tiling_on_sc=False,
                                           needs_layout_passes=False),
      scratch_types=[
          pltpu.VMEM((B,), jnp.int32),
          pltpu.VMEM((RPW, GW), jnp.float32),
          pltpu.VMEM((1, GW), jnp.float32),
          pltpu.VMEM((B + L,), jnp.int32),
          pltpu.VMEM((B + L,), jnp.int32),
          pltpu.VMEM((BPW + L,), jnp.int32),
          pltpu.VMEM((NE, D), jnp.float32),
          pltpu.VMEM((NE, D), jnp.float32),
          pltpu.VMEM((NE, D), jnp.float32),
          pltpu.VMEM((NE, D), jnp.float32),
          pltpu.VMEM((NE, D), jnp.float32),
          pltpu.VMEM((NE, D), jnp.float32),
          pltpu.VMEM((1, L), jnp.int32),
          pltpu.VMEM((1, L), jnp.int32),
          pltpu.SemaphoreType.DMA,
          pltpu.SemaphoreType.DMA,
      ],
  )
  def k(idx_hbm, tab_hbm, mu_hbm, lv_hbm,
        idx_v, ltab, ubuf, blist, rlist, ulist,
        omu0, olv0, omu1, olv1, umu, ulv, bidx0, bidx1, sem0, sem1):
    wid = lax.axis_index("s") * NC + lax.axis_index("c")
    lo = wid * RPW
    hi = jnp.minimum(lo + RPW, R)
    rows = lax.iota(jnp.int32, L)

    pltpu.sync_copy(idx_hbm, idx_v)
    pltpu.sync_copy(tab_hbm.at[pl.ds(lo, RPW)], ltab)
    pltpu.sync_copy(tab_hbm.at[pl.ds(R, 1)], ubuf)

    def scan_seen(i, off):
      v = idx_v[pl.ds(i * L, L)]
      m = jnp.logical_and(v >= lo, v < hi)
      plsc.store_compressed(blist.at[pl.ds(off, L)], rows + i * L, mask=m)
      plsc.store_compressed(rlist.at[pl.ds(off, L)], v - lo, mask=m)
      return off + plsc.all_reduce_population_count(m)[0]

    cnt = lax.fori_loop(0, B // L, scan_seen, 0)
    blist[pl.ds(cnt, L)] = jnp.full((L,), B, jnp.int32)
    rlist[pl.ds(cnt, L)] = jnp.zeros((L,), jnp.int32)

    ubase = wid * BPW
    def scan_unseen(i, off):
      v = idx_v[pl.ds(ubase + i * L, L)]
      m = v == R
      plsc.store_compressed(ulist.at[pl.ds(off, L)], rows + ubase + i * L,
                             mask=m)
      return off + plsc.all_reduce_population_count(m)[0]

    ucnt = lax.fori_loop(0, BPW // L, scan_unseen, 0)
    ulist[pl.ds(ucnt, L)] = jnp.full((L,), B, jnp.int32)

    def moments(src, rl, dst_mu, dst_lv):
      ls = [plsc.load_gather(src, [rl, jnp.full((L,), LOG_OFF + c,
                                                jnp.int32)])
            for c in range(C)]
      mx = ls[0]
      for c in range(1, C):
        mx = jnp.maximum(mx, ls[c])
      es = [jnp.exp(l - mx) for l in ls]
      tot = es[0]
      for c in range(1, C):
        tot = tot + es[c]
      inv = 1.0 / tot
      ws = [e * inv for e in es]

      def dstep(dd, carry):
        dvec = jnp.bitwise_and(rows + dd, D - 1)
        mu_acc = jnp.zeros((L,), jnp.float32)
        sm_acc = jnp.zeros((L,), jnp.float32)
        for c in range(C):
          mu_cd = plsc.load_gather(
              src, [rl, jnp.full((L,), c * D, jnp.int32) + dvec])
          lv_cd = plsc.load_gather(
              src, [rl, jnp.full((L,), C * D + c * D, jnp.int32) + dvec])
          mu_acc = mu_acc + ws[c] * mu_cd
          sm_acc = sm_acc + ws[c] * (jnp.exp(lv_cd) + mu_cd * mu_cd)
        var = jnp.maximum(sm_acc - mu_acc * mu_acc, 1e-6)
        plsc.store_scatter(dst_mu, [rows, dvec], mu_acc)
        plsc.store_scatter(dst_lv, [rows, dvec], _vlog(var))
        return carry

      lax.fori_loop(0, D, dstep, 0)

    moments(ubuf, jnp.zeros((L,), jnp.int32), umu, ulv)

    slots = ((omu0, olv0, bidx0, sem0), (omu1, olv1, bidx1, sem1))

    def fire(dst_mu, dst_lv, bidx, sem, pos_list, j):
      bidx[0] = pos_list[pl.ds(j * L, L)]
      pltpu.async_copy(dst_mu, mu_hbm.at[bidx.at[0]], sem)
      pltpu.async_copy(dst_lv, lv_hbm.at[bidx.at[0]], sem)

    def drain(dst_mu, dst_lv, bidx, sem):
      pltpu.make_async_copy(dst_mu, mu_hbm.at[bidx.at[0]], sem).wait()
      pltpu.make_async_copy(dst_lv, lv_hbm.at[bidx.at[0]], sem).wait()

    nch = (cnt + L - 1) // L

    def seen_pair(o, carry):
      for s in range(2):
        j = o * 2 + s
        dst_mu, dst_lv, bidx, sem = slots[s]

        @pl.when(j < nch)
        def _():
          @pl.when(j >= 2)
          def _():
            drain(dst_mu, dst_lv, bidx, sem)
          rl = rlist[pl.ds(j * L, L)]
          moments(ltab, rl, dst_mu, dst_lv)
          fire(dst_mu, dst_lv, bidx, sem, blist, j)
      return carry

    lax.fori_loop(0, (nch + 1) // 2, seen_pair, 0)
    for s in range(2):
      dst_mu, dst_lv, bidx, sem = slots[s]

      @pl.when(nch > s)
      def _():
        drain(dst_mu, dst_lv, bidx, sem)

    nuch = (ucnt + L - 1) // L

    def unseen_pair(o, carry):
      for s in range(2):
        j = o * 2 + s
        _, _, bidx, sem = slots[s]

        @pl.when(j < nuch)
        def _():
          @pl.when(j >= 2)
          def _():
            drain(umu, ulv, bidx, sem)
          fire(umu, ulv, bidx, sem, ulist, j)
      return carry

    lax.fori_loop(0, (nuch + 1) // 2, unseen_pair, 0)
    for s in range(2):
      _, _, bidx, sem = slots[s]

      @pl.when(nuch > s)
      def _():
        drain(umu, ulv, bidx, sem)

  return k(idx, tab)


def kernel(regime_id, regime_seen_mask, logits_emb, mu_emb, logvar_emb,
           logits_unknown, mu_unknown, logvar_unknown):
  rid = jnp.clip(regime_id, 0, R - 1).astype(jnp.int32)
  idx = jnp.where(regime_seen_mask, rid, R).astype(jnp.int32)
  pad = jnp.full((R + 1, GW - LOG_OFF - C), -1e30, jnp.float32)
  tab = jnp.concatenate([
      jnp.concatenate([mu_emb, mu_unknown.reshape(1, C * D)], 0),
      jnp.concatenate([logvar_emb, logvar_unknown.reshape(1, C * D)], 0),
      jnp.concatenate([logits_emb, logits_unknown.reshape(1, C)], 0),
      pad,
  ], axis=1)
  tab = jnp.concatenate(
      [tab, jnp.zeros((RP - (R + 1), GW), jnp.float32)], axis=0)
  mu, logvar = _sc_fused(idx, tab)
  return (mu[:B], logvar[:B])


# speedup vs baseline: 2.1989x; 1.0608x over previous
import functools

import jax
import jax.numpy as jnp
from jax import lax
from jax.experimental import pallas as pl
from jax.experimental.pallas import tpu as pltpu
from jax.experimental.pallas import tpu_sc as plsc

R, C, D, B = 1000, 8, 128, 16384
GW = 2 * C * D + 16
LOG_OFF = 2 * C * D
RP = 1024

NC, NS, L = 2, 16, 16
NW = NC * NS
RPW = RP // NW
BPW = B // NW
NE = 16

_LN2 = 0.6931471805599453


def _vlog(x):
  bits = plsc.bitcast(x, jnp.int32)
  e = jnp.right_shift(bits, 23) - 127
  m = plsc.bitcast(
      jnp.bitwise_or(jnp.bitwise_and(bits, 0x007FFFFF), 0x3F800000),
      jnp.float32)
  big = m > 1.4142135623730951
  m = jnp.where(big, m * 0.5, m)
  e = jnp.where(big, e + 1, e)
  s = (m - 1.0) / (m + 1.0)
  s2 = s * s
  p = 2.0 * s * (1.0 + s2 * (1.0 / 3.0 + s2 * (0.2 + s2 * (1.0 / 7.0))))
  return e.astype(jnp.float32) * _LN2 + p


def _sc_fused(idx, tab):
  mesh = plsc.VectorSubcoreMesh(core_axis_name="c", subcore_axis_name="s")

  @functools.partial(
      pl.kernel,
      out_type=[
          jax.ShapeDtypeStruct((B, D), jnp.float32),
          jax.ShapeDtypeStruct((B, D), jnp.float32),
      ],
      mesh=mesh,
      compiler_params=pltpu.CompilerParams(use_tc_tiling_on_sc=False,
                                           needs_layout_passes=False),
      scratch_types=[
          pltpu.VMEM((B,), jnp.int32),
          pltpu.VMEM((RPW, GW), jnp.float32),
          pltpu.VMEM((1, GW), jnp.float32),
          pltpu.VMEM((B + L,), jnp.int32),
          pltpu.VMEM((B + L,), jnp.int32),
          pltpu.VMEM((BPW + L,), jnp.int32),
          pltpu.VMEM((NE, D), jnp.float32),
          pltpu.VMEM((NE, D), jnp.float32),
          pltpu.VMEM((NE, D), jnp.float32),
          pltpu.VMEM((NE, D), jnp.float32),
          pltpu.VMEM((NE, D), jnp.float32),
          pltpu.VMEM((NE, D), jnp.float32),
          pltpu.VMEM((1, L), jnp.int32),
          pltpu.VMEM((1, L), jnp.int32),
          pltpu.SemaphoreType.DMA,
          pltpu.SemaphoreType.DMA,
      ],
  )
  def k(idx_hbm, tab_hbm, mu_hbm, lv_hbm,
        idx_v, ltab, ubuf, blist, rlist, ulist,
        omu0, olv0, omu1, olv1, umu, ulv, bidx0, bidx1, sem0, sem1):
    wid = lax.axis_index("s") * NC + lax.axis_index("c")
    lo = wid * RPW
    hi = jnp.minimum(lo + RPW, R)
    rows = lax.iota(jnp.int32, L)

    pltpu.sync_copy(idx_hbm, idx_v)
    pltpu.sync_copy(tab_hbm.at[pl.ds(lo, RPW)], ltab)
    pltpu.sync_copy(tab_hbm.at[pl.ds(R, 1)], ubuf)

    def scan_seen(i, off):
      v = idx_v[pl.ds(i * L, L)]
      m = jnp.logical_and(v >= lo, v < hi)
      plsc.store_compressed(blist.at[pl.ds(off, L)], rows + i * L, mask=m)
      plsc.store_compressed(rlist.at[pl.ds(off, L)], v - lo, mask=m)
      return off + plsc.all_reduce_population_count(m)[0]

    cnt = lax.fori_loop(0, B // L, scan_seen, 0)
    blist[pl.ds(cnt, L)] = plsc.load_gather(
        blist, [jnp.zeros((L,), jnp.int32)])
    rlist[pl.ds(cnt, L)] = plsc.load_gather(
        rlist, [jnp.zeros((L,), jnp.int32)])

    ubase = wid * BPW
    def scan_unseen(i, off):
      v = idx_v[pl.ds(ubase + i * L, L)]
      m = v == R
      plsc.store_compressed(ulist.at[pl.ds(off, L)], rows + ubase + i * L,
                             mask=m)
      return off + plsc.all_reduce_population_count(m)[0]

    ucnt = lax.fori_loop(0, BPW // L, scan_unseen, 0)
    ulist[pl.ds(ucnt, L)] = plsc.load_gather(
        ulist, [jnp.zeros((L,), jnp.int32)])

    def moments(src, rl, dst_mu, dst_lv):
      ls = [plsc.load_gather(src, [rl, jnp.full((L,), LOG_OFF + c,
                                                jnp.int32)])
            for c in range(C)]
      mx = ls[0]
      for c in range(1, C):
        mx = jnp.maximum(mx, ls[c])
      es = [jnp.exp(l - mx) for l in ls]
      tot = es[0]
      for c in range(1, C):
        tot = tot + es[c]
      inv = 1.0 / tot
      ws = [e * inv for e in es]

      def dstep(dd, carry):
        dvec = jnp.bitwise_and(rows + dd, D - 1)
        mu_acc = jnp.zeros((L,), jnp.float32)
        sm_acc = jnp.zeros((L,), jnp.float32)
        for c in range(C):
          mu_cd = plsc.load_gather(
              src, [rl, jnp.full((L,), c * D, jnp.int32) + dvec])
          lv_cd = plsc.load_gather(
              src, [rl, jnp.full((L,), C * D + c * D, jnp.int32) + dvec])
          mu_acc = mu_acc + ws[c] * mu_cd
          sm_acc = sm_acc + ws[c] * (jnp.exp(lv_cd) + mu_cd * mu_cd)
        var = jnp.maximum(sm_acc - mu_acc * mu_acc, 1e-6)
        plsc.store_scatter(dst_mu, [rows, dvec], mu_acc)
        plsc.store_scatter(dst_lv, [rows, dvec], _vlog(var))
        return carry

      lax.fori_loop(0, D, dstep, 0)

    moments(ubuf, jnp.zeros((L,), jnp.int32), umu, ulv)

    slots = ((omu0, olv0, bidx0, sem0), (omu1, olv1, bidx1, sem1))

    def fire(dst_mu, dst_lv, bidx, sem, pos_list, j):
      bidx[0] = pos_list[pl.ds(j * L, L)]
      pltpu.async_copy(dst_mu, mu_hbm.at[bidx.at[0]], sem)
      pltpu.async_copy(dst_lv, lv_hbm.at[bidx.at[0]], sem)

    def drain(dst_mu, dst_lv, bidx, sem):
      pltpu.make_async_copy(dst_mu, mu_hbm.at[bidx.at[0]], sem).wait()
      pltpu.make_async_copy(dst_lv, lv_hbm.at[bidx.at[0]], sem).wait()

    nch = (cnt + L - 1) // L

    def seen_pair(o, carry):
      for s in range(2):
        j = o * 2 + s
        dst_mu, dst_lv, bidx, sem = slots[s]

        @pl.when(j < nch)
        def _():
          @pl.when(j >= 2)
          def _():
            drain(dst_mu, dst_lv, bidx, sem)
          rl = rlist[pl.ds(j * L, L)]
          moments(ltab, rl, dst_mu, dst_lv)
          fire(dst_mu, dst_lv, bidx, sem, blist, j)
      return carry

    lax.fori_loop(0, (nch + 1) // 2, seen_pair, 0)
    for s in range(2):
      dst_mu, dst_lv, bidx, sem = slots[s]

      @pl.when(nch > s)
      def _():
        drain(dst_mu, dst_lv, bidx, sem)

    nuch = (ucnt + L - 1) // L

    def unseen_pair(o, carry):
      for s in range(2):
        j = o * 2 + s
        _, _, bidx, sem = slots[s]

        @pl.when(j < nuch)
        def _():
          @pl.when(j >= 2)
          def _():
            drain(umu, ulv, bidx, sem)
          fire(umu, ulv, bidx, sem, ulist, j)
      return carry

    lax.fori_loop(0, (nuch + 1) // 2, unseen_pair, 0)
    for s in range(2):
      _, _, bidx, sem = slots[s]

      @pl.when(nuch > s)
      def _():
        drain(umu, ulv, bidx, sem)

  return k(idx, tab)


def kernel(regime_id, regime_seen_mask, logits_emb, mu_emb, logvar_emb,
           logits_unknown, mu_unknown, logvar_unknown):
  rid = jnp.clip(regime_id, 0, R - 1).astype(jnp.int32)
  idx = jnp.where(regime_seen_mask, rid, R).astype(jnp.int32)
  pad = jnp.full((R + 1, GW - LOG_OFF - C), -1e30, jnp.float32)
  tab = jnp.concatenate([
      jnp.concatenate([mu_emb, mu_unknown.reshape(1, C * D)], 0),
      jnp.concatenate([logvar_emb, logvar_unknown.reshape(1, C * D)], 0),
      jnp.concatenate([logits_emb, logits_unknown.reshape(1, C)], 0),
      pad,
  ], axis=1)
  tab = jnp.concatenate(
      [tab, jnp.zeros((RP - (R + 1), GW), jnp.float32)], axis=0)
  mu, logvar = _sc_fused(idx, tab)
  return (mu, logvar)
